# Initial kernel scaffold; baseline (speedup 1.0000x reference)
import jax
import jax.numpy as jnp
from jax.experimental import pallas as pl


def kernel(x, adj, num_graphs, in_batch, cluster, W1, b1, W2, b2, W3, b3, Wt1, bt1, Wt2, bt2, W4, b4, W5, b5, Wf1, bf1, Wf2, bf2, Wf3, bf3):
    raise NotImplementedError("write your pallas kernel here")


# speedup vs baseline: 3.5926x; 3.5926x over previous
import jax
import jax.numpy as jnp
from jax.experimental import pallas as pl

N = 10000
CN = 50
NG = 10


def _elu(x):
    return jnp.where(x > 0, x, jnp.expm1(x))


def _mm_kernel(a_ref, w_ref, b_ref, o_ref):
    o_ref[...] = jnp.dot(a_ref[...], w_ref[...],
                         preferred_element_type=jnp.float32) + b_ref[...]


def _mm(a, w, b):
    m, k = a.shape
    n = w.shape[1]
    blk = 1000
    return pl.pallas_call(
        _mm_kernel,
        grid=(m // blk,),
        in_specs=[pl.BlockSpec((blk, k), lambda i: (i, 0)),
                  pl.BlockSpec((k, n), lambda i: (0, 0)),
                  pl.BlockSpec((1, n), lambda i: (0, 0))],
        out_specs=pl.BlockSpec((blk, n), lambda i: (i, 0)),
        out_shape=jax.ShapeDtypeStruct((m, n), jnp.float32),
    )(a, w, b.reshape(1, -1))


def kernel(x, adj, num_graphs, in_batch, cluster, W1, b1, W2, b2, W3, b3,
           Wt1, bt1, Wt2, bt2, W4, b4, W5, b5, Wf1, bf1, Wf2, bf2, Wf3, bf3):
    src, dst = adj[0], adj[1]
    deg = jnp.zeros((N,), jnp.float32).at[dst].add(1.0) + 1.0
    dinv = 1.0 / jnp.sqrt(deg)

    def conv(u, W, b):
        c = _mm(u, W, jnp.zeros((W.shape[1],), jnp.float32))
        g = dinv[:, None] * c
        s = jnp.zeros((N, 128), jnp.float32).at[dst].add(g[src])
        return dinv[:, None] * (s + g) + b

    h = _elu(conv(x, W1, b1))
    h = _elu(conv(h, W2, b2))
    h3 = conv(h, W3, b3)

    mean = h3.mean(axis=0)
    var = (h3 * h3).mean(axis=0) - mean * mean
    rstd = 1.0 / jnp.sqrt(var + 1e-5)

    seg = in_batch * CN + cluster
    S = NG * CN
    counts = jnp.zeros((S,), jnp.float32).at[seg].add(1.0)
    segsum = jnp.zeros((S, 128), jnp.float32).at[seg].add(h3)
    px = (segsum - counts[:, None] * mean[None, :]) \
        / jnp.maximum(counts, 1.0)[:, None] * rstd[None, :]

    ps = seg[src]
    pd = seg[dst]
    Mcnt = jnp.zeros((S, S), jnp.float32).at[ps, pd].add(1.0)
    M = jnp.where(Mcnt > 0, 1.0, 0.0)
    M = M * (1.0 - jnp.eye(S, dtype=jnp.float32))
    degc = M.sum(axis=0) + 1.0
    dinvc = 1.0 / jnp.sqrt(degc)

    def cconv(u, W, b):
        v = dinvc[:, None] * (u @ W)
        t = jax.lax.dot_general(M, v, (((0,), (0,)), ((), ())))
        return dinvc[:, None] * (t + v) + b

    z = _elu(px @ Wt1 + bt1)
    z = _elu(z @ Wt2 + bt2)
    z = _elu(cconv(z, W4, b4))
    z = _elu(cconv(z, W5, b5))
    z = z @ Wf1 + bf1
    k = z.reshape(-1, CN)
    k = _elu(_mm(k, Wf2, bf2))
    k = _mm(k, Wf3, bf3)
    return k


# speedup vs baseline: 4.3872x; 1.2212x over previous
import functools

import jax
import jax.numpy as jnp
from jax import lax
from jax.experimental import pallas as pl
from jax.experimental.pallas import tpu as pltpu
from jax.experimental.pallas import tpu_sc as plsc

N = 10000
E = 320000
CN = 50
NG = 10
SP = 512

_NC, _NS = 2, 16
_NW = _NC * _NS
_EROWS = 80
_EPT = _EROWS * 128
_EPAD = _NW * _EPT
_ACC_ROWS = 10240
_RPT = _ACC_ROWS // _NS

_sc_mesh = plsc.VectorSubcoreMesh(core_axis_name="c", subcore_axis_name="s",
                                  num_cores=_NC, num_subcores=_NS)


def _zero_rows(buf, rows, cols):
    def zrow(i, _):
        for l in range(cols // 16):
            buf[i, pl.ds(l * 16, 16)] = jnp.zeros((16,), jnp.float32)
        return 0
    lax.fori_loop(0, rows, zrow, 0)


_ER = _EPAD // _NS // 128
_PH0 = 2432
_PH1 = 1792


def _edge_body(g_hbm, src_hbm, dst_hbm, out_hbm,
               src_v, dst_v, dstl_v, rows_a, rows_b, zbuf, acc, sem_a, sem_b):
    c = lax.axis_index("c")
    s = lax.axis_index("s")

    _zero_rows(zbuf, 128, 128)
    pltpu.sync_copy(src_hbm.at[pl.ds(s * _ER, _ER)], src_v)
    pltpu.sync_copy(dst_hbm.at[pl.ds(s * _ER, _ER)], dst_v)

    def do_phase(base, win, rpt):
        off = s * rpt
        left = rpt
        while left > 0:
            n = min(128, left)
            pltpu.sync_copy(zbuf.at[pl.ds(0, n)],
                            acc.at[pl.ds(off + rpt - left, n)])
            left -= n
        def remap(j, _):
            for l in range(8):
                dv = dst_v[j, pl.ds(l * 16, 16)]
                loc = dv - base
                valid = (loc >= 0) & (loc < win)
                dstl_v[j, pl.ds(l * 16, 16)] = jnp.where(valid, loc, win)
            return 0
        lax.fori_loop(0, _ER, remap, 0)
        plsc.subcore_barrier()

        pltpu.async_copy(g_hbm.at[src_v.at[0]], rows_a, sem_a)

        def pair(jj, _):
            j = jj * 2
            pltpu.make_async_copy(g_hbm.at[src_v.at[j]], rows_a, sem_a).wait()
            pltpu.async_copy(g_hbm.at[src_v.at[j + 1]], rows_b, sem_b)
            pltpu.sync_copy(rows_a, acc.at[dstl_v.at[j]], add=True)
            pltpu.make_async_copy(g_hbm.at[src_v.at[j + 1]],
                                  rows_b, sem_b).wait()

            @pl.when(jj < _ER // 2 - 1)
            def _():
                pltpu.async_copy(g_hbm.at[src_v.at[j + 2]], rows_a, sem_a)

            pltpu.sync_copy(rows_b, acc.at[dstl_v.at[j + 1]], add=True)
            return 0

        lax.fori_loop(0, _ER // 2, pair, 0)
        plsc.subcore_barrier()
        pltpu.sync_copy(acc.at[pl.ds(s * rpt, rpt)],
                        out_hbm.at[pl.ds(base + s * rpt, rpt)])
        plsc.subcore_barrier()

    @pl.when(c == 0)
    def _():
        do_phase(0, _PH0, _PH0 // _NS)
        do_phase(_PH0, _PH0, _PH0 // _NS)

    @pl.when(c == 1)
    def _():
        do_phase(2 * _PH0, _PH1, _PH1 // _NS)
        do_phase(2 * _PH0 + _PH1, _PH1, _PH1 // _NS)
        do_phase(2 * _PH0 + 2 * _PH1, _PH1, _PH1 // _NS)


_edge_pass = functools.partial(
    pl.kernel,
    out_type=jax.ShapeDtypeStruct((_ACC_ROWS, 128), jnp.float32),
    mesh=_sc_mesh,
    scratch_types=[
        pltpu.VMEM((_ER, 128), jnp.int32),
        pltpu.VMEM((_ER, 128), jnp.int32),
        pltpu.VMEM((_ER, 128), jnp.int32),
        pltpu.VMEM((128, 128), jnp.float32),
        pltpu.VMEM((128, 128), jnp.float32),
        pltpu.VMEM((128, 128), jnp.float32),
        pltpu.VMEM_SHARED((2440, 128), jnp.float32),
        pltpu.SemaphoreType.DMA,
        pltpu.SemaphoreType.DMA,
    ],
)(_edge_body)


_MWORDS = SP * SP
_MPT = _MWORDS // _NS


def _stats_body(src_hbm, dst_hbm, seg_hbm, deg_hbm, segc_hbm, mcnt_hbm,
                src_v, dst_v, code_v, segown_v, ps_v, pd_v, ones_v, zv,
                sem_g, accdeg, accsegc, accm):
    c = lax.axis_index("c")
    s = lax.axis_index("s")
    w = c * _NS + s

    def zo(i, _):
        zv[pl.ds(i * 16, 16)] = jnp.zeros((16,), jnp.float32)
        return 0
    lax.fori_loop(0, 128, zo, 0)
    for l in range(8):
        ones_v[pl.ds(l * 16, 16)] = jnp.ones((16,), jnp.float32)

    pltpu.sync_copy(zv.at[pl.ds(0, 640)], accdeg.at[pl.ds(s * 640, 640)])
    for k in range(_MPT // 2048):
        pltpu.sync_copy(zv, accm.at[pl.ds(s * _MPT + k * 2048, 2048)])

    @pl.when(s == 0)
    def _():
        pltpu.sync_copy(zv.at[pl.ds(0, SP)], accsegc)

    pltpu.sync_copy(src_hbm.at[pl.ds(w * _EROWS, _EROWS)], src_v)
    pltpu.sync_copy(dst_hbm.at[pl.ds(w * _EROWS, _EROWS)], dst_v)
    pltpu.sync_copy(seg_hbm.at[pl.ds(w * 320, 320)], segown_v)
    plsc.subcore_barrier()

    def degj(j, _):
        pltpu.sync_copy(ones_v, accdeg.at[dst_v.at[j]], add=True)
        return 0
    lax.fori_loop(0, _EROWS, degj, 0)

    def segj(k, _):
        idx = segown_v[pl.ds(k * 16, 16)]
        pltpu.sync_copy(ones_v.at[pl.ds(0, 16)], accsegc.at[idx], add=True)
        return 0
    lax.fori_loop(0, 20, segj, 0)

    def codej(j, _):
        pltpu.async_copy(seg_hbm.at[src_v.at[j]], ps_v, sem_g).wait()
        pltpu.async_copy(seg_hbm.at[dst_v.at[j]], pd_v, sem_g).wait()
        for l in range(8):
            ps = ps_v[pl.ds(l * 16, 16)]
            pd = pd_v[pl.ds(l * 16, 16)]
            code_v[j, pl.ds(l * 16, 16)] = ps * SP + pd
        return 0
    lax.fori_loop(0, _EROWS, codej, 0)

    def mj(j, _):
        pltpu.sync_copy(ones_v, accm.at[code_v.at[j]], add=True)
        return 0
    lax.fori_loop(0, _EROWS, mj, 0)

    plsc.subcore_barrier()
    pltpu.sync_copy(accdeg.at[pl.ds(s * 640, 640)],
                    deg_hbm.at[pl.ds(c * _ACC_ROWS + s * 640, 640)])
    pltpu.sync_copy(accm.at[pl.ds(s * _MPT, _MPT)],
                    mcnt_hbm.at[pl.ds(c * _MWORDS + s * _MPT, _MPT)])

    @pl.when(s == 0)
    def _():
        pltpu.sync_copy(accsegc, segc_hbm.at[pl.ds(c * SP, SP)])


_stats_pass = functools.partial(
    pl.kernel,
    out_type=(jax.ShapeDtypeStruct((_NC * _ACC_ROWS,), jnp.float32),
              jax.ShapeDtypeStruct((_NC * SP,), jnp.float32),
              jax.ShapeDtypeStruct((_NC * _MWORDS,), jnp.float32)),
    mesh=_sc_mesh,
    scratch_types=[
        pltpu.VMEM((_EROWS, 128), jnp.int32),
        pltpu.VMEM((_EROWS, 128), jnp.int32),
        pltpu.VMEM((_EROWS, 128), jnp.int32),
        pltpu.VMEM((320,), jnp.int32),
        pltpu.VMEM((128,), jnp.int32),
        pltpu.VMEM((128,), jnp.int32),
        pltpu.VMEM((128,), jnp.float32),
        pltpu.VMEM((2048,), jnp.float32),
        pltpu.SemaphoreType.DMA,
        pltpu.VMEM_SHARED((_ACC_ROWS,), jnp.float32),
        pltpu.VMEM_SHARED((SP,), jnp.float32),
        pltpu.VMEM_SHARED((_MWORDS,), jnp.float32),
    ],
)(_stats_body)


def _pool_body(h_hbm, seg_hbm, pool_hbm, rows_v, seg_v, zbuf, accp):
    c = lax.axis_index("c")
    s = lax.axis_index("s")
    w = c * _NS + s

    _zero_rows(zbuf, 32, 128)
    pltpu.sync_copy(zbuf, accp.at[pl.ds(s * 32, 32)])
    pltpu.sync_copy(seg_hbm.at[pl.ds(w * 320, 320)], seg_v)
    plsc.subcore_barrier()

    def chunk(k, _):
        pltpu.sync_copy(h_hbm.at[pl.ds(w * 320 + k * 64, 64)], rows_v)
        for t in range(4):
            idx = seg_v[pl.ds(k * 64 + t * 16, 16)]
            pltpu.sync_copy(rows_v.at[pl.ds(t * 16, 16)],
                            accp.at[idx], add=True)
        return 0
    lax.fori_loop(0, 5, chunk, 0)

    plsc.subcore_barrier()
    pltpu.sync_copy(accp.at[pl.ds(s * 32, 32)],
                    pool_hbm.at[c, pl.ds(s * 32, 32)])


_pool_pass = functools.partial(
    pl.kernel,
    out_type=jax.ShapeDtypeStruct((_NC, SP, 128), jnp.float32),
    mesh=_sc_mesh,
    scratch_types=[
        pltpu.VMEM((64, 128), jnp.float32),
        pltpu.VMEM((320,), jnp.int32),
        pltpu.VMEM((32, 128), jnp.float32),
        pltpu.VMEM_SHARED((SP, 128), jnp.float32),
    ],
)(_pool_body)


def _elu(x):
    return jnp.where(x > 0, x, jnp.expm1(x))


def _mm_kernel(a_ref, w_ref, b_ref, o_ref):
    o_ref[...] = jnp.dot(a_ref[...], w_ref[...],
                         preferred_element_type=jnp.float32) + b_ref[...]


def _mm(a, w, b):
    m, k = a.shape
    n = w.shape[1]
    blk = 1000
    return pl.pallas_call(
        _mm_kernel,
        grid=(m // blk,),
        in_specs=[pl.BlockSpec((blk, k), lambda i: (i, 0)),
                  pl.BlockSpec((k, n), lambda i: (0, 0)),
                  pl.BlockSpec((1, n), lambda i: (0, 0))],
        out_specs=pl.BlockSpec((blk, n), lambda i: (i, 0)),
        out_shape=jax.ShapeDtypeStruct((m, n), jnp.float32),
    )(a, w, b.reshape(1, -1))


def kernel(x, adj, num_graphs, in_batch, cluster, W1, b1, W2, b2, W3, b3,
           Wt1, bt1, Wt2, bt2, W4, b4, W5, b5, Wf1, bf1, Wf2, bf2, Wf3, bf3):
    src, dst = adj[0], adj[1]
    src2d = jnp.concatenate(
        [src, jnp.zeros((_EPAD - E,), src.dtype)]).reshape(_EPAD // 128, 128)
    dst2d = jnp.concatenate(
        [dst, jnp.full((_EPAD - E,), N, dst.dtype)]).reshape(_EPAD // 128, 128)
    seg_pad = jnp.concatenate(
        [in_batch * CN + cluster,
         jnp.full((_ACC_ROWS - N,), NG * CN, jnp.int32)])

    deg1, segc1, mcnt1 = _stats_pass(src2d, dst2d, seg_pad)
    deg2 = deg1.reshape(_NC, _ACC_ROWS)
    segc2 = segc1.reshape(_NC, SP)
    mcnt2 = mcnt1.reshape(_NC, _MWORDS)
    deg = deg2[0, :N] + deg2[1, :N] + 1.0
    dinv = 1.0 / jnp.sqrt(deg)

    def conv(u, W, b):
        cm = _mm(u, W, jnp.zeros((128,), jnp.float32))
        g = dinv[:, None] * cm
        sp = _edge_pass(g, src2d, dst2d)
        return dinv[:, None] * (sp[:N] + g) + b

    h = _elu(conv(x, W1, b1))
    h = _elu(conv(h, W2, b2))
    h3 = conv(h, W3, b3)

    mean = h3.mean(axis=0)
    var = (h3 * h3).mean(axis=0) - mean * mean
    rstd = 1.0 / jnp.sqrt(var + 1e-5)

    h3pad = jnp.concatenate([h3, jnp.zeros((_ACC_ROWS - N, 128), jnp.float32)])
    pool2 = _pool_pass(h3pad, seg_pad)
    segsum = pool2[0] + pool2[1]
    counts = segc2[0] + segc2[1]
    px = (segsum - counts[:, None] * mean[None, :]) \
        / jnp.maximum(counts, 1.0)[:, None] * rstd[None, :]

    mc = (mcnt2[0] + mcnt2[1]).reshape(SP, SP)
    row_ids = lax.broadcasted_iota(jnp.int32, (SP, SP), 0)
    col_ids = lax.broadcasted_iota(jnp.int32, (SP, SP), 1)
    ok = (mc > 0) & (row_ids != col_ids) \
        & (row_ids < NG * CN) & (col_ids < NG * CN)
    M = jnp.where(ok, 1.0, 0.0)
    degc = M.sum(axis=0) + 1.0
    dinvc = 1.0 / jnp.sqrt(degc)

    def cconv(u, W, b):
        v = dinvc[:, None] * (u @ W)
        t = lax.dot_general(M, v, (((0,), (0,)), ((), ())))
        return dinvc[:, None] * (t + v) + b

    z = _elu(px @ Wt1 + bt1)
    z = _elu(z @ Wt2 + bt2)
    z = _elu(cconv(z, W4, b4))
    z = _elu(cconv(z, W5, b5))
    z = z @ Wf1 + bf1
    k = z[:NG * CN].reshape(-1, CN)
    k = _elu(_mm(k, Wf2, bf2))
    k = _mm(k, Wf3, bf3)
    return k
